# Initial kernel scaffold; baseline (speedup 1.0000x reference)
import jax
import jax.numpy as jnp
from jax.experimental import pallas as pl


def kernel(x_0, x_gt, v_pred, alpha_pred, t):
    raise NotImplementedError("write your pallas kernel here")


# speedup vs baseline: 2.3695x; 2.3695x over previous
import functools

import jax
import jax.numpy as jnp
from jax.experimental import pallas as pl
from jax.experimental.pallas import tpu as pltpu

_REG_OT = 0.1
_SINKHORN_ITERS = 50
_SURVIVAL_THRESHOLD = 1e-05


def _loss_kernel(x0_ref, xgtT_ref, vp_ref, al_ref,
                 lt_ref, lv_ref, ls_ref, sr_ref,
                 km_scr, u_scr, v_scr, acc,
                 *, B, M, K):
    b = pl.program_id(0)
    x0 = x0_ref[0]
    xgtT = xgtT_ref[0]

    C = ((x0[:, 0:1] - xgtT[0:1, :]) ** 2
         + (x0[:, 1:2] - xgtT[1:2, :]) ** 2
         + (x0[:, 2:3] - xgtT[2:3, :]) ** 2)
    cmax = jnp.max(C)
    km_scr[...] = jnp.exp(-(C / (cmax + 1e-12)) / _REG_OT)

    inv_m = jnp.float32(1.0 / M)
    inv_k = jnp.float32(1.0 / K)
    u_scr[...] = jnp.full((M, 1), inv_m, dtype=jnp.float32)
    v_scr[...] = jnp.full((1, K), inv_k, dtype=jnp.float32)

    def body(i, _):
        km = km_scr[...]
        kv = jnp.sum(km * v_scr[...], axis=1, keepdims=True)
        u_scr[...] = inv_m / (kv + 1e-16)
        ktu = jnp.sum(km * u_scr[...], axis=0, keepdims=True)
        v_scr[...] = inv_k / (ktu + 1e-16)
        return 0

    jax.lax.fori_loop(0, _SINKHORN_ITERS, body, 0, unroll=False)

    u = u_scr[...]
    v = v_scr[...]
    pi = (u * km_scr[...]) * v
    rowmax = jnp.max(pi, axis=1, keepdims=True)
    lane = jax.lax.broadcasted_iota(jnp.int32, (M, K), 1)
    jidx = jnp.min(jnp.where(pi == rowmax, lane, K), axis=1, keepdims=True)
    onehot = (lane == jidx).astype(jnp.float32)

    s = (rowmax > _SURVIVAL_THRESHOLD).astype(jnp.float32)

    vel = jnp.zeros((M, 1), dtype=jnp.float32)
    for d in range(3):
        matched_d = jnp.sum(onehot * xgtT[d:d + 1, :], axis=1, keepdims=True)
        vt_d = matched_d - x0[:, d:d + 1]
        diff_d = vp_ref[0][:, d:d + 1] - vt_d
        vel = vel + diff_d * diff_d
    vel_b = jnp.sum(s * vel)
    s_b = jnp.sum(s)

    z = al_ref[0]
    bce_b = jnp.sum(jnp.maximum(z, 0.0) - z * s
                    + jnp.log1p(jnp.exp(-jnp.abs(z))))

    @pl.when(b == 0)
    def _init():
        acc[0] = s_b
        acc[1] = vel_b
        acc[2] = bce_b

    @pl.when(b > 0)
    def _accum():
        acc[0] = acc[0] + s_b
        acc[1] = acc[1] + vel_b
        acc[2] = acc[2] + bce_b

    @pl.when(b == B - 1)
    def _finalize():
        s_tot = acc[0]
        num_surv = jnp.maximum(s_tot, 1.0)
        loss_vel = acc[1] / num_surv
        loss_surv = acc[2] / jnp.float32(B * M)
        lv_ref[...] = loss_vel.reshape(1, 1)
        ls_ref[...] = loss_surv.reshape(1, 1)
        lt_ref[...] = (loss_vel + loss_surv).reshape(1, 1)
        sr_ref[...] = (s_tot / jnp.float32(B * M)).reshape(1, 1)


def kernel(x_0, x_gt, v_pred, alpha_pred, t):
    B, M, _ = x_0.shape
    K = x_gt.shape[1]

    pad3 = lambda a: jnp.pad(a, ((0, 0), (0, 0), (0, 5)))
    x0p = pad3(x_0)
    vpp = pad3(v_pred)
    xgtT = jnp.pad(jnp.transpose(x_gt, (0, 2, 1)),
                   ((0, 0), (0, 5), (0, 0)))

    out_shapes = [jax.ShapeDtypeStruct((1, 1), jnp.float32)] * 4
    scalar_spec = pl.BlockSpec((1, 1), lambda b: (0, 0))

    outs = pl.pallas_call(
        functools.partial(_loss_kernel, B=B, M=M, K=K),
        grid=(B,),
        in_specs=[
            pl.BlockSpec((1, M, 8), lambda b: (b, 0, 0)),
            pl.BlockSpec((1, 8, K), lambda b: (b, 0, 0)),
            pl.BlockSpec((1, M, 8), lambda b: (b, 0, 0)),
            pl.BlockSpec((1, M, 1), lambda b: (b, 0, 0)),
        ],
        out_specs=[scalar_spec] * 4,
        out_shape=out_shapes,
        scratch_shapes=[
            pltpu.VMEM((M, K), jnp.float32),
            pltpu.VMEM((M, 1), jnp.float32),
            pltpu.VMEM((1, K), jnp.float32),
            pltpu.SMEM((3,), jnp.float32),
        ],
        compiler_params=pltpu.CompilerParams(
            dimension_semantics=("arbitrary",),
        ),
    )(x0p, xgtT, vpp, alpha_pred)

    lt, lv, ls, sr = (o.reshape(()) for o in outs)
    return (lt, lv, ls, sr)
